# baseline (speedup 1.0000x reference)
import jax
import jax.numpy as jnp
from jax.experimental import pallas as pl

_TEMPERATURE = 64.0


def _protonet_body(s_ref, q_ref, o_ref):
    inv_t = 1.0 / _TEMPERATURE
    proto = jnp.sum(s_ref[...], axis=0) * (1.0 / s_ref.shape[0])
    q = q_ref[...]
    pn = (jnp.sum(proto * proto, axis=1) * inv_t)[None, :]
    cross = jax.lax.dot_general(
        q, proto * (2.0 * inv_t), (((1,), (1,)), ((), ())),
        preferred_element_type=jnp.float32,
    )
    ones_t = jnp.full((q.shape[1], pn.shape[1]), inv_t, jnp.float32)
    qn_mat = jax.lax.dot_general(
        q * q, ones_t, (((1,), (0,)), ((), ())),
        preferred_element_type=jnp.float32,
    )
    o_ref[...] = cross - qn_mat - pn


def kernel(support, query):
    n_batch, n_shot, n_way, emb_dim = support.shape
    n_query = n_batch * query.shape[1] * n_way
    s = support.reshape(n_shot, n_way, emb_dim)
    q = query.reshape(n_query, emb_dim)
    return pl.pallas_call(
        _protonet_body,
        out_shape=jax.ShapeDtypeStruct((n_query, n_way), jnp.float32),
    )(s, q)
